# baseline (speedup 1.0000x reference)
import jax
import jax.numpy as jnp
from jax.experimental import pallas as pl

B, C, H, W = 32, 768, 32, 32
HW = H * W


def _select_body(m_ref, f_ref, t_ref, o_ref):
    m = m_ref[0] != 0
    f = f_ref[0]
    t = t_ref[0]
    o_ref[0] = jnp.where(m, f, t)


def kernel(features, active_mask, mask_token):
    f3 = features.reshape(B, C, HW)
    m3 = active_mask.astype(jnp.int32).reshape(B, 1, HW)
    t3 = mask_token.reshape(1, C, 1)
    out = pl.pallas_call(
        _select_body,
        grid=(B,),
        in_specs=[
            pl.BlockSpec((1, 1, HW), lambda b: (b, 0, 0)),
            pl.BlockSpec((1, C, HW), lambda b: (b, 0, 0)),
            pl.BlockSpec((1, C, 1), lambda b: (0, 0, 0)),
        ],
        out_specs=pl.BlockSpec((1, C, HW), lambda b: (b, 0, 0)),
        out_shape=jax.ShapeDtypeStruct((B, C, HW), jnp.float32),
    )(m3, f3, t3)
    return out.reshape(B, C, H, W)
